# baseline (speedup 1.0000x reference)
import functools

import numpy as np

import jax
import jax.numpy as jnp
from jax import lax
from jax.experimental import pallas as pl
from jax.experimental.pallas import tpu as pltpu
from jax.experimental.pallas import tpu_sc as plsc

F32 = jnp.float32


def _agg_stride2(Wh):
    tap = {(0, 0): -1, (1, 0): 0, (0, 1): 1}
    zeros = jnp.zeros_like(Wh[0, 0])
    rows = []
    for sr in (0, 1):
        cols = []
        for sc in (0, 1):
            blocks = []
            for pr in (0, 1):
                for pc in (0, 1):
                    dy = tap.get((pr, sr))
                    dx = tap.get((pc, sc))
                    blocks.append(zeros if dy is None or dx is None
                                  else Wh[dy + 1, dx + 1])
            cols.append(jnp.concatenate(blocks, axis=0))
        rows.append(jnp.stack(cols))
    return jnp.stack(rows)


_SETS = {'T': ((-1, (0,)), (0, (1, 2))),
         'M': ((0, (0, 1, 2)),),
         'B': ((0, (0, 1)), (1, (2,)))}
_CH2CV = {'L': 'T', 'M': 'M', 'R': 'B'}


def _build_decoder_terms(Wc1):
    Wtap = jnp.transpose(Wc1, (2, 3, 1, 0))
    mats, case_terms = [], []
    for cv in 'TMB':
        for ch in 'LMR':
            terms = []
            for (sy, kys) in _SETS[cv]:
                for (sx, kxs) in _SETS[_CH2CV[ch]]:
                    K = sum(Wtap[ky, kx] for ky in kys for kx in kxs)
                    terms.append((sy, sx, len(mats)))
                    mats.append(K)
            case_terms.append(terms)
    return jnp.stack(mats), case_terms


def _build_gg():
    gg = np.zeros((3, 196, 3136), np.float32)
    for gh in range(14):
        for gw in range(14):
            g = gh * 14 + gw
            base = gh * 224 + gw * 16
            gg[0, g, base] = 1.0
            gg[1, g, base + 1:base + 15] = 1.0
            gg[2, g, base + 15] = 1.0
    return gg


_GG_NP = _build_gg()


def _parity_pack(a, n):
    r = a.reshape(n, 2, n, 2, a.shape[-1])
    return jnp.concatenate(
        [r[:, 0, :, 0, :], r[:, 0, :, 1, :],
         r[:, 1, :, 0, :], r[:, 1, :, 1, :]], axis=-1)


def _enc_body(xw_ref, wagg1_ref, b1_ref, wagg2_ref, b2_ref, wp_ref, bp_ref,
              out_ref):
    xw = xw_ref[0]
    rr = xw.reshape(113, 2, 678)
    re = rr[:, 0, :].reshape(113, 113, 6)
    ro = rr[:, 1, :].reshape(113, 113, 6)
    acc = jnp.zeros((12544, 96), F32)
    for sr in (0, 1):
        for sc in (0, 1):
            q = jnp.concatenate(
                [re[sr:sr + 112, sc:sc + 112, :],
                 ro[sr:sr + 112, sc:sc + 112, :]], axis=-1).reshape(12544, 12)
            acc = acc + jnp.dot(q, wagg1_ref[sr, sc],
                                preferred_element_type=F32)
    y = jnp.maximum(acc + b1_ref[0], 0.0).reshape(112, 112, 96)
    p2 = _parity_pack(jnp.pad(y, ((1, 1), (1, 1), (0, 0))), 57)
    acc2 = jnp.zeros((3136, 96), F32)
    for sr in (0, 1):
        for sc in (0, 1):
            sl = p2[sr:sr + 56, sc:sc + 56, :].reshape(3136, 384)
            acc2 = acc2 + jnp.dot(sl, wagg2_ref[sr, sc],
                                  preferred_element_type=F32)
    h = jnp.maximum(acc2 + b2_ref[0], 0.0).reshape(14, 4, 14, 4, 96)
    pooled = (h.sum(axis=3).sum(axis=1) * 0.0625).reshape(196, 96)
    out_ref[0] = jnp.dot(pooled, wp_ref[...],
                         preferred_element_type=F32) + bp_ref[0]


def _vq_body(z_ref, embt_ref, emb_ref, wgp_ref, bgp_ref,
             idx_ref, loss_ref, vc_ref):
    z = z_ref[...]
    embt = embt_ref[...]
    z2 = jnp.sum(z * z, axis=1, keepdims=True)
    e2 = jnp.sum(embt * embt, axis=0, keepdims=True)
    ze = jnp.dot(z, embt, preferred_element_type=F32)
    dist = z2 + e2 - 2.0 * ze
    mn = jnp.min(dist, axis=1, keepdims=True)
    ids = lax.broadcasted_iota(jnp.int32, dist.shape, 1)
    idx_col = jnp.min(jnp.where(dist == mn, ids, 1024),
                      axis=1, keepdims=True)
    idx_ref[...] = jnp.pad(idx_col.reshape(8, 196), ((0, 0), (0, 60)))
    loss_ref[...] = (1.25 * jnp.sum(mn) / (1568.0 * 64.0)).reshape(1, 1)
    vc_ref[...] = jnp.maximum(
        jnp.dot(emb_ref[...], wgp_ref[...], preferred_element_type=F32)
        + bgp_ref[0], 0.0)


def _make_dec_body(case_terms):
    def _dec_body(v_ref, ak_ref, bc1_ref, wc2_ref, bc2_ref, gg_ref, out_ref):
        v3 = v_ref[0][:196, :96].reshape(14, 14, 96)
        vpad = jnp.pad(v3, ((1, 1), (1, 1), (0, 0)))
        shifted = {}
        for sy in (-1, 0, 1):
            for sx in (-1, 0, 1):
                shifted[(sy, sx)] = (
                    vpad[1 + sy:15 + sy, 1 + sx:15 + sx, :].reshape(196, 96))
        maps = []
        for terms in case_terms:
            acc = jnp.zeros((196, 96), F32)
            for (sy, sx, t) in terms:
                acc = acc + jnp.dot(shifted[(sy, sx)], ak_ref[t],
                                    preferred_element_type=F32)
            hc = jnp.maximum(acc + bc1_ref[0], 0.0)
            mp = jnp.dot(hc, wc2_ref[...],
                         preferred_element_type=F32) + bc2_ref[0]
            maps.append(mp.T)
        rows = []
        for cvi in range(3):
            r = jnp.zeros((3, 3136), F32)
            for chi in range(3):
                r = r + jnp.dot(maps[cvi * 3 + chi], gg_ref[chi],
                                preferred_element_type=F32)
            rows.append(r.reshape(3, 14, 224))
        row_t, row_m, row_b = rows
        img = jnp.concatenate(
            [row_t[:, :, None, :],
             jnp.broadcast_to(row_m[:, :, None, :], (3, 14, 14, 224)),
             row_b[:, :, None, :]], axis=2).reshape(3, 224, 224)
        out_ref[0] = img
    return _dec_body


def _sc_gather(vcodes, idx_pad):
    mesh = plsc.VectorSubcoreMesh(core_axis_name="c", subcore_axis_name="s")

    @functools.partial(
        pl.kernel, mesh=mesh,
        out_type=jax.ShapeDtypeStruct((2048, 128), F32),
        scratch_types=[
            pltpu.VMEM((64,), jnp.int32),
            pltpu.VMEM((64, 128), F32),
            pltpu.SemaphoreType.DMA,
        ])
    def k(table_hbm, idx_hbm, out_hbm, idx_v, rows_v, sem):
        wid = lax.axis_index("s") * 2 + lax.axis_index("c")
        base = wid * 64
        pltpu.sync_copy(idx_hbm.at[pl.ds(base, 64)], idx_v)
        pltpu.async_copy(table_hbm.at[idx_v], rows_v, sem).wait()
        pltpu.sync_copy(rows_v, out_hbm.at[pl.ds(base, 64)])

    return k(vcodes, idx_pad)


def kernel(x, W1, b1, W2, b2, Wp, bp, emb, Wgp, bgp, Wc1, bc1, Wc2, bc2):
    x_nhwc = jnp.transpose(x, (0, 2, 3, 1))
    xp = jnp.pad(x_nhwc, ((0, 0), (1, 1), (1, 1), (0, 0)))
    xw = xp.reshape(8, 226, 678)
    wagg1 = _agg_stride2(jnp.transpose(W1, (2, 3, 1, 0)))
    wagg2 = _agg_stride2(jnp.transpose(W2, (2, 3, 1, 0)))
    z = pl.pallas_call(
        _enc_body,
        grid=(8,),
        in_specs=[
            pl.BlockSpec((1, 226, 678), lambda b: (b, 0, 0)),
            pl.BlockSpec((2, 2, 12, 96), lambda b: (0, 0, 0, 0)),
            pl.BlockSpec((1, 96), lambda b: (0, 0)),
            pl.BlockSpec((2, 2, 384, 96), lambda b: (0, 0, 0, 0)),
            pl.BlockSpec((1, 96), lambda b: (0, 0)),
            pl.BlockSpec((96, 64), lambda b: (0, 0)),
            pl.BlockSpec((1, 64), lambda b: (0, 0)),
        ],
        out_specs=pl.BlockSpec((1, 196, 64), lambda b: (b, 0, 0)),
        out_shape=jax.ShapeDtypeStruct((8, 196, 64), F32),
    )(xw, wagg1, b1.reshape(1, 96), wagg2, b2.reshape(1, 96),
      Wp[:, :, 0, 0].T, bp.reshape(1, 64))
    z_flat = z.reshape(1568, 64)

    idx2d, loss11, vcodes = pl.pallas_call(
        _vq_body,
        out_shape=[
            jax.ShapeDtypeStruct((8, 256), jnp.int32),
            jax.ShapeDtypeStruct((1, 1), F32),
            jax.ShapeDtypeStruct((1024, 128), F32),
        ],
    )(z_flat, emb.T, emb,
      jnp.pad(Wgp[:, :, 0, 0].T, ((0, 0), (0, 32))),
      jnp.pad(bgp.reshape(1, 96), ((0, 0), (0, 32))))
    idx = idx2d[:, :196].reshape(8, 14, 14)

    vg = _sc_gather(vcodes, idx2d.reshape(2048)).reshape(8, 256, 128)

    ak, case_terms = _build_decoder_terms(Wc1)
    x_hat = pl.pallas_call(
        _make_dec_body(case_terms),
        grid=(8,),
        in_specs=[
            pl.BlockSpec((1, 256, 128), lambda b: (b, 0, 0)),
            pl.BlockSpec((25, 96, 96), lambda b: (0, 0, 0)),
            pl.BlockSpec((1, 96), lambda b: (0, 0)),
            pl.BlockSpec((96, 3), lambda b: (0, 0)),
            pl.BlockSpec((1, 3), lambda b: (0, 0)),
            pl.BlockSpec((3, 196, 3136), lambda b: (0, 0, 0)),
        ],
        out_specs=pl.BlockSpec((1, 3, 224, 224), lambda b: (b, 0, 0, 0)),
        out_shape=jax.ShapeDtypeStruct((8, 3, 224, 224), F32),
    )(vg, ak, bc1.reshape(1, 96), Wc2[:, :, 0, 0].T, bc2.reshape(1, 3),
      jnp.asarray(_GG_NP))

    return x_hat, idx, loss11.reshape(())


# speedup vs baseline: 1.0798x; 1.0798x over previous
import functools

import numpy as np

import jax
import jax.numpy as jnp
from jax import lax
from jax.experimental import pallas as pl
from jax.experimental.pallas import tpu as pltpu
from jax.experimental.pallas import tpu_sc as plsc

F32 = jnp.float32


def _agg_stride2(Wh):
    tap = {(0, 0): -1, (1, 0): 0, (0, 1): 1}
    zeros = jnp.zeros_like(Wh[0, 0])
    rows = []
    for sr in (0, 1):
        cols = []
        for sc in (0, 1):
            blocks = []
            for pr in (0, 1):
                for pc in (0, 1):
                    dy = tap.get((pr, sr))
                    dx = tap.get((pc, sc))
                    blocks.append(zeros if dy is None or dx is None
                                  else Wh[dy + 1, dx + 1])
            cols.append(jnp.concatenate(blocks, axis=0))
        rows.append(jnp.stack(cols))
    return jnp.stack(rows)


_SETS = {'T': ((-1, (0,)), (0, (1, 2))),
         'M': ((0, (0, 1, 2)),),
         'B': ((0, (0, 1)), (1, (2,)))}
_CH2CV = {'L': 'T', 'M': 'M', 'R': 'B'}


def _build_decoder_terms(Wc1):
    Wtap = jnp.transpose(Wc1, (2, 3, 1, 0))
    mats, case_terms = [], []
    for cv in 'TMB':
        for ch in 'LMR':
            terms = []
            for (sy, kys) in _SETS[cv]:
                for (sx, kxs) in _SETS[_CH2CV[ch]]:
                    K = sum(Wtap[ky, kx] for ky in kys for kx in kxs)
                    terms.append((sy, sx, len(mats)))
                    mats.append(K)
            case_terms.append(terms)
    return jnp.stack(mats), case_terms


def _build_gg():
    gg = np.zeros((3, 196, 3136), np.float32)
    for gh in range(14):
        for gw in range(14):
            g = gh * 14 + gw
            base = gh * 224 + gw * 16
            gg[0, g, base] = 1.0
            gg[1, g, base + 1:base + 15] = 1.0
            gg[2, g, base + 15] = 1.0
    return gg


_GG_NP = _build_gg()


def _build_interleave():
    e = np.zeros((3, 224, 678), np.float32)
    for ic in range(3):
        for c in range(224):
            e[ic, c, 3 * (c + 1) + ic] = 1.0
    return e


_EIL_NP = _build_interleave()


def _parity_pack(a, n):
    r = a.reshape(n, 2, n, 2, a.shape[-1])
    return jnp.concatenate(
        [r[:, 0, :, 0, :], r[:, 0, :, 1, :],
         r[:, 1, :, 0, :], r[:, 1, :, 1, :]], axis=-1)


def _enc1_body(x_ref, eil_ref, wagg_ref, b1_ref, out_ref):
    xb = x_ref[0]
    xm = jnp.zeros((224, 678), F32)
    for ic in range(3):
        xm = xm + jnp.dot(xb[ic], eil_ref[ic], preferred_element_type=F32)
    xw = jnp.pad(xm, ((1, 1), (0, 0)))
    rr = xw.reshape(113, 2, 678)
    re = rr[:, 0, :].reshape(113, 113, 6)
    ro = rr[:, 1, :].reshape(113, 113, 6)
    acc = jnp.zeros((12544, 96), F32)
    for sr in (0, 1):
        for sc in (0, 1):
            q = jnp.concatenate(
                [re[sr:sr + 112, sc:sc + 112, :],
                 ro[sr:sr + 112, sc:sc + 112, :]], axis=-1).reshape(12544, 12)
            acc = acc + jnp.dot(q, wagg_ref[sr, sc],
                                preferred_element_type=F32)
    y = jnp.maximum(acc + b1_ref[0], 0.0)
    out_ref[0] = y.reshape(112, 112, 96)


def _enc2_body(y_ref, wagg_ref, b2_ref, wp_ref, bp_ref, out_ref):
    yb = y_ref[0]
    p2 = _parity_pack(jnp.pad(yb, ((1, 1), (1, 1), (0, 0))), 57)
    acc2 = jnp.zeros((3136, 96), F32)
    for sr in (0, 1):
        for sc in (0, 1):
            sl = p2[sr:sr + 56, sc:sc + 56, :].reshape(3136, 384)
            acc2 = acc2 + jnp.dot(sl, wagg_ref[sr, sc],
                                  preferred_element_type=F32)
    h = jnp.maximum(acc2 + b2_ref[0], 0.0).reshape(14, 4, 14, 4, 96)
    pooled = (h.sum(axis=3).sum(axis=1) * 0.0625).reshape(196, 96)
    out_ref[0] = jnp.dot(pooled, wp_ref[...],
                         preferred_element_type=F32) + bp_ref[0]


def _vq_body(z_ref, embt_ref, emb_ref, wgp_ref, bgp_ref,
             idx_ref, loss_ref, vc_ref):
    z = z_ref[...]
    embt = embt_ref[...]
    z2 = jnp.sum(z * z, axis=1, keepdims=True)
    e2 = jnp.sum(embt * embt, axis=0, keepdims=True)
    ze = jnp.dot(z, embt, preferred_element_type=F32)
    dist = z2 + e2 - 2.0 * ze
    mn = jnp.min(dist, axis=1, keepdims=True)
    ids = lax.broadcasted_iota(jnp.int32, dist.shape, 1)
    idx_col = jnp.min(jnp.where(dist == mn, ids, 1024),
                      axis=1, keepdims=True)
    idx_ref[...] = jnp.pad(idx_col.reshape(8, 196), ((0, 0), (0, 60)))
    loss_ref[...] = (1.25 * jnp.sum(mn) / (1568.0 * 64.0)).reshape(1, 1)
    vc_ref[...] = jnp.maximum(
        jnp.dot(emb_ref[...], wgp_ref[...], preferred_element_type=F32)
        + bgp_ref[0], 0.0)


def _make_dec_body(case_terms):
    def _dec_body(v_ref, ak_ref, bc1_ref, wc2_ref, bc2_ref, gg_ref, out_ref):
        v3 = v_ref[0][:196, :96].reshape(14, 14, 96)
        vpad = jnp.pad(v3, ((1, 1), (1, 1), (0, 0)))
        shifted = {}
        for sy in (-1, 0, 1):
            for sx in (-1, 0, 1):
                shifted[(sy, sx)] = (
                    vpad[1 + sy:15 + sy, 1 + sx:15 + sx, :].reshape(196, 96))
        maps = []
        for terms in case_terms:
            acc = jnp.zeros((196, 96), F32)
            for (sy, sx, t) in terms:
                acc = acc + jnp.dot(shifted[(sy, sx)], ak_ref[t],
                                    preferred_element_type=F32)
            hc = jnp.maximum(acc + bc1_ref[0], 0.0)
            mp = jnp.dot(hc, wc2_ref[...],
                         preferred_element_type=F32) + bc2_ref[0]
            maps.append(mp.T)
        rows = []
        for cvi in range(3):
            r = jnp.zeros((3, 3136), F32)
            for chi in range(3):
                r = r + jnp.dot(maps[cvi * 3 + chi], gg_ref[chi],
                                preferred_element_type=F32)
            rows.append(r.reshape(3, 14, 224))
        row_t, row_m, row_b = rows
        img = jnp.concatenate(
            [row_t[:, :, None, :],
             jnp.broadcast_to(row_m[:, :, None, :], (3, 14, 14, 224)),
             row_b[:, :, None, :]], axis=2).reshape(3, 224, 224)
        out_ref[0] = img
    return _dec_body


def _sc_gather(vcodes, idx_pad):
    mesh = plsc.VectorSubcoreMesh(core_axis_name="c", subcore_axis_name="s")

    @functools.partial(
        pl.kernel, mesh=mesh,
        out_type=jax.ShapeDtypeStruct((2048, 128), F32),
        scratch_types=[
            pltpu.VMEM((64,), jnp.int32),
            pltpu.VMEM((64, 128), F32),
            pltpu.SemaphoreType.DMA,
        ])
    def k(table_hbm, idx_hbm, out_hbm, idx_v, rows_v, sem):
        wid = lax.axis_index("s") * 2 + lax.axis_index("c")
        base = wid * 64
        pltpu.sync_copy(idx_hbm.at[pl.ds(base, 64)], idx_v)
        pltpu.async_copy(table_hbm.at[idx_v], rows_v, sem).wait()
        pltpu.sync_copy(rows_v, out_hbm.at[pl.ds(base, 64)])

    return k(vcodes, idx_pad)


def kernel(x, W1, b1, W2, b2, Wp, bp, emb, Wgp, bgp, Wc1, bc1, Wc2, bc2):
    wagg1 = _agg_stride2(jnp.transpose(W1, (2, 3, 1, 0)))
    y1 = pl.pallas_call(
        _enc1_body,
        grid=(8,),
        in_specs=[
            pl.BlockSpec((1, 3, 224, 224), lambda b: (b, 0, 0, 0)),
            pl.BlockSpec((3, 224, 678), lambda b: (0, 0, 0)),
            pl.BlockSpec((2, 2, 12, 96), lambda b: (0, 0, 0, 0)),
            pl.BlockSpec((1, 96), lambda b: (0, 0)),
        ],
        out_specs=pl.BlockSpec((1, 112, 112, 96), lambda b: (b, 0, 0, 0)),
        out_shape=jax.ShapeDtypeStruct((8, 112, 112, 96), F32),
    )(x, jnp.asarray(_EIL_NP), wagg1, b1.reshape(1, 96))

    wagg2 = _agg_stride2(jnp.transpose(W2, (2, 3, 1, 0)))
    z = pl.pallas_call(
        _enc2_body,
        grid=(8,),
        in_specs=[
            pl.BlockSpec((1, 112, 112, 96), lambda b: (b, 0, 0, 0)),
            pl.BlockSpec((2, 2, 384, 96), lambda b: (0, 0, 0, 0)),
            pl.BlockSpec((1, 96), lambda b: (0, 0)),
            pl.BlockSpec((96, 64), lambda b: (0, 0)),
            pl.BlockSpec((1, 64), lambda b: (0, 0)),
        ],
        out_specs=pl.BlockSpec((1, 196, 64), lambda b: (b, 0, 0)),
        out_shape=jax.ShapeDtypeStruct((8, 196, 64), F32),
    )(y1, wagg2, b2.reshape(1, 96), Wp[:, :, 0, 0].T, bp.reshape(1, 64))
    z_flat = z.reshape(1568, 64)

    idx2d, loss11, vcodes = pl.pallas_call(
        _vq_body,
        out_shape=[
            jax.ShapeDtypeStruct((8, 256), jnp.int32),
            jax.ShapeDtypeStruct((1, 1), F32),
            jax.ShapeDtypeStruct((1024, 128), F32),
        ],
    )(z_flat, emb.T, emb,
      jnp.pad(Wgp[:, :, 0, 0].T, ((0, 0), (0, 32))),
      jnp.pad(bgp.reshape(1, 96), ((0, 0), (0, 32))))
    idx = idx2d[:, :196].reshape(8, 14, 14)

    vg = _sc_gather(vcodes, idx2d.reshape(2048)).reshape(8, 256, 128)

    ak, case_terms = _build_decoder_terms(Wc1)
    x_hat = pl.pallas_call(
        _make_dec_body(case_terms),
        grid=(8,),
        in_specs=[
            pl.BlockSpec((1, 256, 128), lambda b: (b, 0, 0)),
            pl.BlockSpec((25, 96, 96), lambda b: (0, 0, 0)),
            pl.BlockSpec((1, 96), lambda b: (0, 0)),
            pl.BlockSpec((96, 3), lambda b: (0, 0)),
            pl.BlockSpec((1, 3), lambda b: (0, 0)),
            pl.BlockSpec((3, 196, 3136), lambda b: (0, 0, 0)),
        ],
        out_specs=pl.BlockSpec((1, 3, 224, 224), lambda b: (b, 0, 0, 0)),
        out_shape=jax.ShapeDtypeStruct((8, 3, 224, 224), F32),
    )(vg, ak, bc1.reshape(1, 96), Wc2[:, :, 0, 0].T, bc2.reshape(1, 3),
      jnp.asarray(_GG_NP))

    return x_hat, idx, loss11.reshape(())
